# Initial kernel scaffold; baseline (speedup 1.0000x reference)
import jax
import jax.numpy as jnp
from jax.experimental import pallas as pl


def kernel(grid_nfeat, mesh_ndata, g2m_edata, mesh_edata, m2g_edata, params, g2m_src, g2m_dst, mesh_src, mesh_dst, m2g_src, m2g_dst):
    raise NotImplementedError("write your pallas kernel here")


# speedup vs baseline: 1.0259x; 1.0259x over previous
import jax
import jax.numpy as jnp
from jax.experimental import pallas as pl

HID = 512


def _ident_tc(x):
    def body(x_ref, o_ref):
        o_ref[...] = x_ref[...]
    return pl.pallas_call(
        body,
        out_shape=jax.ShapeDtypeStruct(x.shape, x.dtype),
    )(x)


def _mlp(p, x, ln=True):
    h = jax.nn.silu(x @ p["W0"] + p["b0"])
    y = h @ p["W1"] + p["b1"]
    if ln:
        mu = jnp.mean(y, axis=-1, keepdims=True)
        var = jnp.var(y, axis=-1, keepdims=True)
        y = (y - mu) / jnp.sqrt(var + 1e-5) * p["gamma"] + p["beta"]
    return y


def _mlp_pre(p, pre):
    h = jax.nn.silu(pre + p["b0"])
    y = h @ p["W1"] + p["b1"]
    mu = jnp.mean(y, axis=-1, keepdims=True)
    var = jnp.var(y, axis=-1, keepdims=True)
    return (y - mu) / jnp.sqrt(var + 1e-5) * p["gamma"] + p["beta"]


def _edge(p, e, stab, sidx, dtab, didx):
    W0 = p["W0"]
    pre = e @ W0[:HID] + (stab @ W0[HID:2 * HID])[sidx] + (dtab @ W0[2 * HID:])[didx]
    return _mlp_pre(p, pre) + e


def _node(p, agg, x):
    W0 = p["W0"]
    pre = agg @ W0[:HID] + x @ W0[HID:]
    return _mlp_pre(p, pre) + x


def kernel(grid_nfeat, mesh_ndata, g2m_edata, mesh_edata, m2g_edata, params,
           g2m_src, g2m_dst, mesh_src, mesh_dst, m2g_src, m2g_dst):
    n_mesh = mesh_ndata.shape[0]
    n_grid = grid_nfeat.shape[0]
    grid = _mlp(params["emb_grid"], grid_nfeat)
    mesh = _mlp(params["emb_mesh"], mesh_ndata)
    e_g2m = _mlp(params["emb_g2m"], g2m_edata)
    e_mesh = _mlp(params["emb_meshe"], mesh_edata)

    e_g2m = _edge(params["enc_edge"], e_g2m, grid, g2m_src, mesh, g2m_dst)
    agg = jax.ops.segment_sum(e_g2m, g2m_dst, num_segments=n_mesh)
    mesh = _node(params["enc_mesh"], agg, mesh)
    grid_enc = _mlp(params["enc_grid"], grid) + grid

    for blk in params["proc"]:
        e_mesh = _edge(blk["edge"], e_mesh, mesh, mesh_src, mesh, mesh_dst)
        agg = jax.ops.segment_sum(e_mesh, mesh_dst, num_segments=n_mesh)
        mesh = _node(blk["node"], agg, mesh)

    e_m2g = _mlp(params["dec_emb"], m2g_edata)
    e_m2g = _edge(params["dec_edge"], e_m2g, mesh, m2g_src, grid_enc, m2g_dst)
    agg = jax.ops.segment_sum(e_m2g, m2g_dst, num_segments=n_grid)
    grid_dec = _node(params["dec_node"], agg, grid_enc)
    out = _mlp(params["finale"], grid_dec, ln=False)
    return _ident_tc(out)


# speedup vs baseline: 1.2982x; 1.2655x over previous
import functools

import jax
import jax.numpy as jnp
from jax import lax
from jax.experimental import pallas as pl
from jax.experimental.pallas import tpu as pltpu
from jax.experimental.pallas import tpu_sc as plsc

HID = 512
NC = 2
NS = 16
NW = NC * NS
CHUNK = 80
EDGE_ALIGN = NW * CHUNK
IDXC = 1024

_BIG = 2 ** 30


def _pad_rows(x, n):
    if x.shape[0] == n:
        return x
    return jnp.pad(x, ((0, n - x.shape[0]),) + ((0, 0),) * (x.ndim - 1))


def _pad2(x, n, k):
    return jnp.pad(x, ((0, n - x.shape[0]), (0, k - x.shape[1])))


def _pad_idx(idx, n, fill):
    if idx.shape[0] == n:
        return idx.astype(jnp.int32)
    return jnp.concatenate(
        [idx.astype(jnp.int32), jnp.full((n - idx.shape[0],), fill, jnp.int32)])


def _mlp_tc(xs, Ws, adds, vecs, W1, res, *, ln, bn, dout):
    n = xs[0].shape[0]
    grid = n // bn
    nx, na = len(xs), len(adds)
    has_res = res is not None

    def body(*refs):
        i = 0
        x_refs = refs[i:i + nx]; i += nx
        w_refs = refs[i:i + nx]; i += nx
        a_refs = refs[i:i + na]; i += na
        vecs_ref = refs[i]; i += 1
        w1_ref = refs[i]; i += 1
        r_ref = refs[i] if has_res else None; i += int(has_res)
        out_ref = refs[i]

        v = vecs_ref[...]
        b0 = v[0:1, :]
        b1 = v[1:2, :dout]
        acc = jnp.dot(x_refs[0][...], w_refs[0][...],
                      preferred_element_type=jnp.float32)
        for xr, wr in zip(x_refs[1:], w_refs[1:]):
            acc = acc + jnp.dot(xr[...], wr[...],
                                preferred_element_type=jnp.float32)
        for ar in a_refs:
            acc = acc + ar[...]
        h = acc + b0
        h = h * jax.nn.sigmoid(h)
        y = jnp.dot(h, w1_ref[...], preferred_element_type=jnp.float32) + b1
        if ln:
            gamma = v[2:3, :dout]
            beta = v[3:4, :dout]
            mu = jnp.mean(y, axis=-1, keepdims=True)
            yc = y - mu
            var = jnp.mean(yc * yc, axis=-1, keepdims=True)
            y = yc * lax.rsqrt(var + 1e-5) * gamma + beta
        if has_res:
            y = y + r_ref[...]
        out_ref[...] = y

    in_specs = (
        [pl.BlockSpec((bn, x.shape[1]), lambda i: (i, 0)) for x in xs]
        + [pl.BlockSpec(w.shape, lambda i: (0, 0)) for w in Ws]
        + [pl.BlockSpec((bn, HID), lambda i: (i, 0)) for _ in adds]
        + [pl.BlockSpec((8, HID), lambda i: (0, 0)),
           pl.BlockSpec(W1.shape, lambda i: (0, 0))]
        + ([pl.BlockSpec((bn, HID), lambda i: (i, 0))] if has_res else [])
    )
    args = list(xs) + list(Ws) + list(adds) + [vecs, W1] + ([res] if has_res else [])
    return pl.pallas_call(
        body,
        grid=(grid,),
        in_specs=in_specs,
        out_specs=pl.BlockSpec((bn, dout), lambda i: (i, 0)),
        out_shape=jax.ShapeDtypeStruct((n, dout), jnp.float32),
    )(*args)


def _vecs(p, dout=HID):
    v = jnp.zeros((8, HID), jnp.float32)
    v = v.at[0, :].set(p["b0"])
    v = v.at[1, :dout].set(p["b1"])
    v = v.at[2, :dout].set(p["gamma"])
    v = v.at[3, :dout].set(p["beta"])
    return v


def _mlp_small(p, x, bn):
    w0 = _pad_rows(p["W0"], 128)
    return _mlp_tc([x], [w0], [], _vecs(p), p["W1"], None,
                   ln=True, bn=bn, dout=HID)


def _proj_tc(x, W):
    n = x.shape[0]
    bn = next(b for b in (512, 384, 256, 128) if n % b == 0)

    def body(x_ref, w_ref, out_ref):
        out_ref[...] = jnp.dot(x_ref[...], w_ref[...],
                               preferred_element_type=jnp.float32)

    return pl.pallas_call(
        body,
        grid=(n // bn,),
        in_specs=[pl.BlockSpec((bn, HID), lambda i: (i, 0)),
                  pl.BlockSpec(W.shape, lambda i: (0, 0))],
        out_specs=pl.BlockSpec((bn, HID), lambda i: (i, 0)),
        out_shape=jax.ShapeDtypeStruct((n, HID), jnp.float32),
    )(x, W)


@functools.lru_cache(maxsize=None)
def _sc_gather_pair_fn(B, V1, V2):
    per_w = B // NW
    k = per_w // CHUNK
    mesh = plsc.VectorSubcoreMesh(core_axis_name="c", subcore_axis_name="s")

    @functools.partial(
        pl.kernel,
        out_type=[jax.ShapeDtypeStruct((B, HID), jnp.float32),
                  jax.ShapeDtypeStruct((B, HID), jnp.float32)],
        mesh=mesh,
        scratch_types=[
            pltpu.VMEM((CHUNK,), jnp.int32),
            pltpu.VMEM((CHUNK, HID), jnp.float32),
            pltpu.VMEM((CHUNK,), jnp.int32),
            pltpu.VMEM((CHUNK, HID), jnp.float32),
            pltpu.SemaphoreType.DMA,
            pltpu.SemaphoreType.DMA,
        ],
    )
    def kern(tab1, idx1, tab2, idx2, out1, out2,
             idx1_v, rows1_v, idx2_v, rows2_v, sem1, sem2):
        wid = lax.axis_index("s") * NC + lax.axis_index("c")
        base = wid * per_w

        def step(i, _):
            off = base + i * CHUNK
            pltpu.sync_copy(idx1.at[pl.ds(off, CHUNK)], idx1_v)
            pltpu.sync_copy(idx2.at[pl.ds(off, CHUNK)], idx2_v)
            cp1 = pltpu.async_copy(tab1.at[idx1_v], rows1_v, sem1)
            cp2 = pltpu.async_copy(tab2.at[idx2_v], rows2_v, sem2)
            cp1.wait()
            cp2.wait()
            pltpu.sync_copy(rows1_v, out1.at[pl.ds(off, CHUNK)])
            pltpu.sync_copy(rows2_v, out2.at[pl.ds(off, CHUNK)])
            return 0

        lax.fori_loop(0, k, step, 0, unroll=False)

    return kern


def _sc_gather_pair(tab1, idx1, tab2, idx2):
    return _sc_gather_pair_fn(idx1.shape[0], tab1.shape[0], tab2.shape[0])(
        tab1, idx1, tab2, idx2)


@functools.lru_cache(maxsize=None)
def _sc_segsum_fn(E, TR, nch):
    n_idxc = E // IDXC
    GB = CHUNK
    CAP = 112
    mesh = plsc.VectorSubcoreMesh(core_axis_name="c", subcore_axis_name="s")

    @functools.partial(
        pl.kernel,
        out_type=jax.ShapeDtypeStruct((nch * NW * TR, HID), jnp.float32),
        mesh=mesh,
        scratch_types=[
            pltpu.VMEM((IDXC,), jnp.int32),
            pltpu.VMEM((CAP,), jnp.int32),
            pltpu.VMEM((CAP,), jnp.int32),
            pltpu.VMEM((GB, HID), jnp.float32),
            pltpu.VMEM((TR, HID), jnp.float32),
            pltpu.SemaphoreType.DMA,
        ],
        compiler_params=pltpu.CompilerParams(needs_layout_passes=False),
    )
    def kern(vals, idx, zeros, out, idx_v, eidx_v, dloc_v, rows_v, acc, sem):
        c = lax.axis_index("c")
        s = lax.axis_index("s")
        t = s * NC + c
        iota = lax.iota(jnp.int32, 16)
        zero16 = jnp.zeros((16,), jnp.int32)

        for q in range(CAP // 16):
            eidx_v[pl.ds(q * 16, 16)] = zero16
            dloc_v[pl.ds(q * 16, 16)] = zero16

        def accum(n):
            pltpu.async_copy(vals.at[eidx_v.at[pl.ds(0, GB)]], rows_v,
                             sem).wait()

            def row_body(r, _):
                g = (r // 16) * 16
                lane = r - g
                dv = dloc_v[pl.ds(g, 16)]
                d = jnp.max(jnp.where(iota == lane, dv, 0))

                def col(j, _):
                    a = acc[d, pl.ds(j * 16, 16)]
                    acc[d, pl.ds(j * 16, 16)] = a + rows_v[r, pl.ds(j * 16, 16)]
                    return 0

                lax.fori_loop(0, HID // 16, col, 0, unroll=True)
                return 0

            lax.fori_loop(0, n, row_body, 0)

        def spill(cur):
            accum(GB)
            er = eidx_v[pl.ds(GB, 16)]
            dr = dloc_v[pl.ds(GB, 16)]
            eidx_v[pl.ds(0, 16)] = er
            dloc_v[pl.ds(0, 16)] = dr
            return cur - GB

        for ch in range(nch):
            lo = (ch * NW + t) * TR
            pltpu.sync_copy(zeros, acc)

            def idx_chunk(ci, cur):
                pltpu.sync_copy(idx.at[pl.ds(ci * IDXC, IDXC)], idx_v)

                def grp(gi, cur):
                    dst16 = idx_v[pl.ds(gi * 16, 16)]
                    m = (dst16 >= lo) & (dst16 < lo + TR)
                    cnt = jnp.sum(jnp.where(m, 1.0, 0.0)).astype(jnp.int32)
                    plsc.store_compressed(
                        eidx_v.at[pl.ds(cur, 16)],
                        ci * IDXC + gi * 16 + iota, mask=m)
                    plsc.store_compressed(
                        dloc_v.at[pl.ds(cur, 16)], dst16 - lo, mask=m)
                    cur = cur + cnt
                    return lax.cond(cur >= GB, spill, lambda x: x, cur)

                return lax.fori_loop(0, IDXC // 16, grp, cur)

            cur = lax.fori_loop(0, n_idxc, idx_chunk, jnp.int32(0))
            accum(cur)
            pltpu.sync_copy(acc, out.at[pl.ds((ch * NW + t) * TR, TR)])

    return kern


def _sc_segsum(vals, idx, TR, nch):
    E = vals.shape[0]
    zeros = jnp.zeros((TR, HID), jnp.float32)
    return _sc_segsum_fn(E, TR, nch)(vals, idx, zeros)


def _edge_stage(p, e, src_tab, src_idx, dst_tab, dst_idx, bn):
    W0 = p["W0"]
    W0e, W0s, W0d = W0[:HID], W0[HID:2 * HID], W0[2 * HID:]
    ps = _proj_tc(src_tab, W0s)
    pd = _proj_tc(dst_tab, W0d)
    gs, gd = _sc_gather_pair(ps, src_idx, pd, dst_idx)
    return _mlp_tc([e], [W0e], [gs, gd], _vecs(p), p["W1"], e,
                   ln=True, bn=bn, dout=HID)


def _node_stage(p, agg, x, bn):
    W0 = p["W0"]
    return _mlp_tc([agg, x], [W0[:HID], W0[HID:]], [], _vecs(p), p["W1"], x,
                   ln=True, bn=bn, dout=HID)


def kernel(grid_nfeat, mesh_ndata, g2m_edata, mesh_edata, m2g_edata, params,
           g2m_src, g2m_dst, mesh_src, mesh_dst, m2g_src, m2g_dst):
    n_grid, n_mesh = grid_nfeat.shape[0], mesh_ndata.shape[0]
    NGP = 10240
    NMP = 2688
    MTR = 88
    GTR = 160
    EG = 40960
    EM = 15360
    ED = 30720

    gridf = _pad2(grid_nfeat, NGP, 128)
    meshf = _pad2(mesh_ndata, NMP, 128)
    g2mf = _pad2(g2m_edata, EG, 128)
    meshef = _pad2(mesh_edata, EM, 128)
    m2gf = _pad2(m2g_edata, ED, 128)

    g2m_src_g = _pad_idx(g2m_src, EG, 0)
    g2m_dst_g = _pad_idx(g2m_dst, EG, 0)
    g2m_dst_s = _pad_idx(g2m_dst, EG, _BIG)
    mesh_src_g = _pad_idx(mesh_src, EM, 0)
    mesh_dst_g = _pad_idx(mesh_dst, EM, 0)
    mesh_dst_s = _pad_idx(mesh_dst, EM, _BIG)
    m2g_src_g = _pad_idx(m2g_src, ED, 0)
    m2g_dst_g = _pad_idx(m2g_dst, ED, 0)
    m2g_dst_s = _pad_idx(m2g_dst, ED, _BIG)

    grid = _mlp_small(params["emb_grid"], gridf, 512)
    mesh = _mlp_small(params["emb_mesh"], meshf, 384)
    e_g2m = _mlp_small(params["emb_g2m"], g2mf, 512)
    e_mesh = _mlp_small(params["emb_meshe"], meshef, 512)

    e_g2m = _edge_stage(params["enc_edge"], e_g2m, grid, g2m_src_g,
                        mesh, g2m_dst_g, 512)
    agg = _sc_segsum(e_g2m, g2m_dst_s, MTR, 1)[:NMP]
    mesh = _node_stage(params["enc_mesh"], agg, mesh, 384)
    grid_enc = _mlp_tc([grid], [params["enc_grid"]["W0"]], [],
                       _vecs(params["enc_grid"]), params["enc_grid"]["W1"],
                       grid, ln=True, bn=512, dout=HID)

    for blk in params["proc"]:
        e_mesh = _edge_stage(blk["edge"], e_mesh, mesh, mesh_src_g,
                             mesh, mesh_dst_g, 512)
        agg = _sc_segsum(e_mesh, mesh_dst_s, MTR, 1)[:NMP]
        mesh = _node_stage(blk["node"], agg, mesh, 384)

    e_m2g = _mlp_small(params["dec_emb"], m2gf, 512)
    e_m2g = _edge_stage(params["dec_edge"], e_m2g, mesh, m2g_src_g,
                        grid_enc, m2g_dst_g, 512)
    agg = _sc_segsum(e_m2g, m2g_dst_s, GTR, 2)
    grid_dec = _node_stage(params["dec_node"], agg, grid_enc, 512)

    pf = params["finale"]
    w1 = jnp.pad(pf["W1"], ((0, 0), (0, 127)))
    vec = jnp.zeros((8, HID), jnp.float32).at[0, :].set(pf["b0"])
    vec = vec.at[1, 0].set(pf["b1"][0])
    out = _mlp_tc([grid_dec], [pf["W0"]], [], vec, w1, None,
                  ln=False, bn=512, dout=128)
    return out[:n_grid, :1]


# speedup vs baseline: 1.3190x; 1.0161x over previous
import functools

import jax
import jax.numpy as jnp
from jax import lax
from jax.experimental import pallas as pl
from jax.experimental.pallas import tpu as pltpu
from jax.experimental.pallas import tpu_sc as plsc

HID = 512
NC = 2
NS = 16
NW = NC * NS
CHUNK = 80
EDGE_ALIGN = NW * CHUNK
IDXC = 1024

_BIG = 2 ** 30


def _pad_rows(x, n):
    if x.shape[0] == n:
        return x
    return jnp.pad(x, ((0, n - x.shape[0]),) + ((0, 0),) * (x.ndim - 1))


def _pad2(x, n, k):
    return jnp.pad(x, ((0, n - x.shape[0]), (0, k - x.shape[1])))


def _pad_idx(idx, n, fill):
    if idx.shape[0] == n:
        return idx.astype(jnp.int32)
    return jnp.concatenate(
        [idx.astype(jnp.int32), jnp.full((n - idx.shape[0],), fill, jnp.int32)])


def _mlp_tc(xs, Ws, adds, vecs, W1, res, *, ln, bn, dout):
    n = xs[0].shape[0]
    grid = n // bn
    nx, na = len(xs), len(adds)
    has_res = res is not None

    def body(*refs):
        i = 0
        x_refs = refs[i:i + nx]; i += nx
        w_refs = refs[i:i + nx]; i += nx
        a_refs = refs[i:i + na]; i += na
        vecs_ref = refs[i]; i += 1
        w1_ref = refs[i]; i += 1
        r_ref = refs[i] if has_res else None; i += int(has_res)
        out_ref = refs[i]

        v = vecs_ref[...]
        b0 = v[0:1, :]
        b1 = v[1:2, :dout]
        acc = jnp.dot(x_refs[0][...], w_refs[0][...],
                      preferred_element_type=jnp.float32)
        for xr, wr in zip(x_refs[1:], w_refs[1:]):
            acc = acc + jnp.dot(xr[...], wr[...],
                                preferred_element_type=jnp.float32)
        for ar in a_refs:
            acc = acc + ar[...]
        h = acc + b0
        h = h * jax.nn.sigmoid(h)
        y = jnp.dot(h, w1_ref[...], preferred_element_type=jnp.float32) + b1
        if ln:
            gamma = v[2:3, :dout]
            beta = v[3:4, :dout]
            mu = jnp.mean(y, axis=-1, keepdims=True)
            yc = y - mu
            var = jnp.mean(yc * yc, axis=-1, keepdims=True)
            y = yc * lax.rsqrt(var + 1e-5) * gamma + beta
        if has_res:
            y = y + r_ref[...]
        out_ref[...] = y

    in_specs = (
        [pl.BlockSpec((bn, x.shape[1]), lambda i: (i, 0)) for x in xs]
        + [pl.BlockSpec(w.shape, lambda i: (0, 0)) for w in Ws]
        + [pl.BlockSpec((bn, HID), lambda i: (i, 0)) for _ in adds]
        + [pl.BlockSpec((8, HID), lambda i: (0, 0)),
           pl.BlockSpec(W1.shape, lambda i: (0, 0))]
        + ([pl.BlockSpec((bn, HID), lambda i: (i, 0))] if has_res else [])
    )
    args = list(xs) + list(Ws) + list(adds) + [vecs, W1] + ([res] if has_res else [])
    return pl.pallas_call(
        body,
        grid=(grid,),
        in_specs=in_specs,
        out_specs=pl.BlockSpec((bn, dout), lambda i: (i, 0)),
        out_shape=jax.ShapeDtypeStruct((n, dout), jnp.float32),
    )(*args)


def _vecs(p, dout=HID):
    v = jnp.zeros((8, HID), jnp.float32)
    v = v.at[0, :].set(p["b0"])
    v = v.at[1, :dout].set(p["b1"])
    v = v.at[2, :dout].set(p["gamma"])
    v = v.at[3, :dout].set(p["beta"])
    return v


def _mlp_small(p, x, bn):
    w0 = _pad_rows(p["W0"], 128)
    return _mlp_tc([x], [w0], [], _vecs(p), p["W1"], None,
                   ln=True, bn=bn, dout=HID)


def _proj_tc(x, W):
    n = x.shape[0]
    bn = next(b for b in (512, 384, 256, 128) if n % b == 0)

    def body(x_ref, w_ref, out_ref):
        out_ref[...] = jnp.dot(x_ref[...], w_ref[...],
                               preferred_element_type=jnp.float32)

    return pl.pallas_call(
        body,
        grid=(n // bn,),
        in_specs=[pl.BlockSpec((bn, HID), lambda i: (i, 0)),
                  pl.BlockSpec(W.shape, lambda i: (0, 0))],
        out_specs=pl.BlockSpec((bn, HID), lambda i: (i, 0)),
        out_shape=jax.ShapeDtypeStruct((n, HID), jnp.float32),
    )(x, W)


@functools.lru_cache(maxsize=None)
def _sc_gather_pair_fn(B, V1, V2):
    per_w = B // NW
    mesh = plsc.VectorSubcoreMesh(core_axis_name="c", subcore_axis_name="s")

    GC = 40
    k = per_w // GC

    @functools.partial(
        pl.kernel,
        out_type=[jax.ShapeDtypeStruct((B, HID), jnp.float32),
                  jax.ShapeDtypeStruct((B, HID), jnp.float32)],
        mesh=mesh,
        scratch_types=[
            pltpu.VMEM((per_w,), jnp.int32),
            pltpu.VMEM((per_w,), jnp.int32),
            pltpu.VMEM((2, GC, HID), jnp.float32),
            pltpu.VMEM((2, GC, HID), jnp.float32),
            pltpu.SemaphoreType.DMA,
            pltpu.SemaphoreType.DMA,
            pltpu.SemaphoreType.DMA,
            pltpu.SemaphoreType.DMA,
            pltpu.SemaphoreType.DMA,
            pltpu.SemaphoreType.DMA,
            pltpu.SemaphoreType.DMA,
            pltpu.SemaphoreType.DMA,
        ],
    )
    def kern(tab1, idx1, tab2, idx2, out1, out2, idx1_v, idx2_v,
             rows1_v, rows2_v, g1a, g1b, g2a, g2b, o1a, o1b, o2a, o2b):
        wid = lax.axis_index("s") * NC + lax.axis_index("c")
        base = wid * per_w
        pltpu.sync_copy(idx1.at[pl.ds(base, per_w)], idx1_v)
        pltpu.sync_copy(idx2.at[pl.ds(base, per_w)], idx2_v)
        gsems = ((g1a, g2a), (g1b, g2b))
        osems = ((o1a, o2a), (o1b, o2b))
        ocps = {}
        for i in range(k):
            b = i & 1
            if i >= 2:
                ocps[(i - 2, 0)].wait()
                ocps[(i - 2, 1)].wait()
            cp1 = pltpu.async_copy(
                tab1.at[idx1_v.at[pl.ds(i * GC, GC)]],
                rows1_v.at[b], gsems[b][0])
            cp2 = pltpu.async_copy(
                tab2.at[idx2_v.at[pl.ds(i * GC, GC)]],
                rows2_v.at[b], gsems[b][1])
            cp1.wait()
            cp2.wait()
            ocps[(i, 0)] = pltpu.async_copy(
                rows1_v.at[b], out1.at[pl.ds(base + i * GC, GC)],
                osems[b][0])
            ocps[(i, 1)] = pltpu.async_copy(
                rows2_v.at[b], out2.at[pl.ds(base + i * GC, GC)],
                osems[b][1])
        for i in range(max(k - 2, 0), k):
            ocps[(i, 0)].wait()
            ocps[(i, 1)].wait()

    return kern


def _sc_gather_pair(tab1, idx1, tab2, idx2):
    return _sc_gather_pair_fn(idx1.shape[0], tab1.shape[0], tab2.shape[0])(
        tab1, idx1, tab2, idx2)


@functools.lru_cache(maxsize=None)
def _sc_segsum_fn(E, TR, nch):
    n_idxc = E // IDXC
    GB = CHUNK
    CAP = 112
    mesh = plsc.VectorSubcoreMesh(core_axis_name="c", subcore_axis_name="s")

    @functools.partial(
        pl.kernel,
        out_type=jax.ShapeDtypeStruct((nch * NW * TR, HID), jnp.float32),
        mesh=mesh,
        scratch_types=[
            pltpu.VMEM((IDXC,), jnp.int32),
            pltpu.VMEM((CAP,), jnp.int32),
            pltpu.VMEM((CAP,), jnp.int32),
            pltpu.VMEM((GB, HID), jnp.float32),
            pltpu.VMEM((TR, HID), jnp.float32),
            pltpu.SemaphoreType.DMA,
        ],
        compiler_params=pltpu.CompilerParams(needs_layout_passes=False),
    )
    def kern(vals, idx, zeros, out, idx_v, eidx_v, dloc_v, rows_v, acc, sem):
        c = lax.axis_index("c")
        s = lax.axis_index("s")
        t = s * NC + c
        iota = lax.iota(jnp.int32, 16)
        zero16 = jnp.zeros((16,), jnp.int32)

        for q in range(CAP // 16):
            eidx_v[pl.ds(q * 16, 16)] = zero16
            dloc_v[pl.ds(q * 16, 16)] = zero16

        def accum(n):
            pltpu.async_copy(vals.at[eidx_v.at[pl.ds(0, GB)]], rows_v,
                             sem).wait()

            def row_body(r, _):
                g = (r // 16) * 16
                lane = r - g
                dv = dloc_v[pl.ds(g, 16)]
                d = jnp.max(jnp.where(iota == lane, dv, 0))

                def col(j, _):
                    a = acc[d, pl.ds(j * 16, 16)]
                    acc[d, pl.ds(j * 16, 16)] = a + rows_v[r, pl.ds(j * 16, 16)]
                    return 0

                lax.fori_loop(0, HID // 16, col, 0, unroll=True)
                return 0

            lax.fori_loop(0, n, row_body, 0)

        def spill(cur):
            accum(GB)
            er = eidx_v[pl.ds(GB, 16)]
            dr = dloc_v[pl.ds(GB, 16)]
            eidx_v[pl.ds(0, 16)] = er
            dloc_v[pl.ds(0, 16)] = dr
            return cur - GB

        for ch in range(nch):
            lo = (ch * NW + t) * TR
            pltpu.sync_copy(zeros, acc)

            def idx_chunk(ci, cur):
                pltpu.sync_copy(idx.at[pl.ds(ci * IDXC, IDXC)], idx_v)

                def grp(gi, cur):
                    dst16 = idx_v[pl.ds(gi * 16, 16)]
                    m = (dst16 >= lo) & (dst16 < lo + TR)
                    cnt = jnp.sum(jnp.where(m, 1.0, 0.0)).astype(jnp.int32)
                    plsc.store_compressed(
                        eidx_v.at[pl.ds(cur, 16)],
                        ci * IDXC + gi * 16 + iota, mask=m)
                    plsc.store_compressed(
                        dloc_v.at[pl.ds(cur, 16)], dst16 - lo, mask=m)
                    cur = cur + cnt
                    return lax.cond(cur >= GB, spill, lambda x: x, cur)

                return lax.fori_loop(0, IDXC // 16, grp, cur)

            cur = lax.fori_loop(0, n_idxc, idx_chunk, jnp.int32(0))
            accum(cur)
            pltpu.sync_copy(acc, out.at[pl.ds((ch * NW + t) * TR, TR)])

    return kern


def _sc_segsum(vals, idx, TR, nch):
    E = vals.shape[0]
    zeros = jnp.zeros((TR, HID), jnp.float32)
    return _sc_segsum_fn(E, TR, nch)(vals, idx, zeros)


def _edge_stage(p, e, src_tab, src_idx, dst_tab, dst_idx, bn):
    W0 = p["W0"]
    W0e, W0s, W0d = W0[:HID], W0[HID:2 * HID], W0[2 * HID:]
    ps = _proj_tc(src_tab, W0s)
    pd = _proj_tc(dst_tab, W0d)
    gs, gd = _sc_gather_pair(ps, src_idx, pd, dst_idx)
    return _mlp_tc([e], [W0e], [gs, gd], _vecs(p), p["W1"], e,
                   ln=True, bn=bn, dout=HID)


def _node_stage(p, agg, x, bn):
    W0 = p["W0"]
    return _mlp_tc([agg, x], [W0[:HID], W0[HID:]], [], _vecs(p), p["W1"], x,
                   ln=True, bn=bn, dout=HID)


def kernel(grid_nfeat, mesh_ndata, g2m_edata, mesh_edata, m2g_edata, params,
           g2m_src, g2m_dst, mesh_src, mesh_dst, m2g_src, m2g_dst):
    n_grid, n_mesh = grid_nfeat.shape[0], mesh_ndata.shape[0]
    NGP = 10240
    NMP = 2688
    MTR = 88
    GTR = 160
    EG = 40960
    EM = 15360
    ED = 30720

    gridf = _pad2(grid_nfeat, NGP, 128)
    meshf = _pad2(mesh_ndata, NMP, 128)
    g2mf = _pad2(g2m_edata, EG, 128)
    meshef = _pad2(mesh_edata, EM, 128)
    m2gf = _pad2(m2g_edata, ED, 128)

    g2m_src_g = _pad_idx(g2m_src, EG, 0)
    g2m_dst_g = _pad_idx(g2m_dst, EG, 0)
    g2m_dst_s = _pad_idx(g2m_dst, EG, _BIG)
    mesh_src_g = _pad_idx(mesh_src, EM, 0)
    mesh_dst_g = _pad_idx(mesh_dst, EM, 0)
    mesh_dst_s = _pad_idx(mesh_dst, EM, _BIG)
    m2g_src_g = _pad_idx(m2g_src, ED, 0)
    m2g_dst_g = _pad_idx(m2g_dst, ED, 0)
    m2g_dst_s = _pad_idx(m2g_dst, ED, _BIG)

    grid = _mlp_small(params["emb_grid"], gridf, 512)
    mesh = _mlp_small(params["emb_mesh"], meshf, 384)
    e_g2m = _mlp_small(params["emb_g2m"], g2mf, 512)
    e_mesh = _mlp_small(params["emb_meshe"], meshef, 512)

    e_g2m = _edge_stage(params["enc_edge"], e_g2m, grid, g2m_src_g,
                        mesh, g2m_dst_g, 512)
    agg = _sc_segsum(e_g2m, g2m_dst_s, MTR, 1)[:NMP]
    mesh = _node_stage(params["enc_mesh"], agg, mesh, 384)
    grid_enc = _mlp_tc([grid], [params["enc_grid"]["W0"]], [],
                       _vecs(params["enc_grid"]), params["enc_grid"]["W1"],
                       grid, ln=True, bn=512, dout=HID)

    for blk in params["proc"]:
        e_mesh = _edge_stage(blk["edge"], e_mesh, mesh, mesh_src_g,
                             mesh, mesh_dst_g, 512)
        agg = _sc_segsum(e_mesh, mesh_dst_s, MTR, 1)[:NMP]
        mesh = _node_stage(blk["node"], agg, mesh, 384)

    e_m2g = _mlp_small(params["dec_emb"], m2gf, 512)
    e_m2g = _edge_stage(params["dec_edge"], e_m2g, mesh, m2g_src_g,
                        grid_enc, m2g_dst_g, 512)
    agg = _sc_segsum(e_m2g, m2g_dst_s, GTR, 2)
    grid_dec = _node_stage(params["dec_node"], agg, grid_enc, 512)

    pf = params["finale"]
    w1 = jnp.pad(pf["W1"], ((0, 0), (0, 127)))
    vec = jnp.zeros((8, HID), jnp.float32).at[0, :].set(pf["b0"])
    vec = vec.at[1, 0].set(pf["b1"][0])
    out = _mlp_tc([grid_dec], [pf["W0"]], [], vec, w1, None,
                  ln=False, bn=512, dout=128)
    return out[:n_grid, :1]


# speedup vs baseline: 1.7153x; 1.3004x over previous
import functools

import jax
import jax.numpy as jnp
from jax import lax
from jax.experimental import pallas as pl
from jax.experimental.pallas import tpu as pltpu
from jax.experimental.pallas import tpu_sc as plsc

HID = 512
NC = 2
NS = 16
NW = NC * NS
CHUNK = 80
EDGE_ALIGN = NW * CHUNK
IDXC = 1024

_BIG = 2 ** 30


def _pad_rows(x, n):
    if x.shape[0] == n:
        return x
    return jnp.pad(x, ((0, n - x.shape[0]),) + ((0, 0),) * (x.ndim - 1))


def _pad2(x, n, k):
    return jnp.pad(x, ((0, n - x.shape[0]), (0, k - x.shape[1])))


def _pad_idx(idx, n, fill):
    if idx.shape[0] == n:
        return idx.astype(jnp.int32)
    return jnp.concatenate(
        [idx.astype(jnp.int32), jnp.full((n - idx.shape[0],), fill, jnp.int32)])


def _mlp_tc(xs, Ws, adds, vecs, W1, res, *, ln, bn, dout):
    n = xs[0].shape[0]
    grid = n // bn
    nx, na = len(xs), len(adds)
    has_res = res is not None

    def body(*refs):
        i = 0
        x_refs = refs[i:i + nx]; i += nx
        w_refs = refs[i:i + nx]; i += nx
        a_refs = refs[i:i + na]; i += na
        vecs_ref = refs[i]; i += 1
        w1_ref = refs[i]; i += 1
        r_ref = refs[i] if has_res else None; i += int(has_res)
        out_ref = refs[i]

        v = vecs_ref[...]
        b0 = v[0:1, :]
        b1 = v[1:2, :dout]
        acc = jnp.dot(x_refs[0][...], w_refs[0][...],
                      preferred_element_type=jnp.float32)
        for xr, wr in zip(x_refs[1:], w_refs[1:]):
            acc = acc + jnp.dot(xr[...], wr[...],
                                preferred_element_type=jnp.float32)
        for ar in a_refs:
            acc = acc + ar[...]
        h = acc + b0
        h = h * jax.nn.sigmoid(h)
        y = jnp.dot(h, w1_ref[...], preferred_element_type=jnp.float32) + b1
        if ln:
            gamma = v[2:3, :dout]
            beta = v[3:4, :dout]
            mu = jnp.mean(y, axis=-1, keepdims=True)
            yc = y - mu
            var = jnp.mean(yc * yc, axis=-1, keepdims=True)
            y = yc * lax.rsqrt(var + 1e-5) * gamma + beta
        if has_res:
            y = y + r_ref[...]
        out_ref[...] = y

    in_specs = (
        [pl.BlockSpec((bn, x.shape[1]), lambda i: (i, 0)) for x in xs]
        + [pl.BlockSpec(w.shape, lambda i: (0, 0)) for w in Ws]
        + [pl.BlockSpec((bn, HID), lambda i: (i, 0)) for _ in adds]
        + [pl.BlockSpec((8, HID), lambda i: (0, 0)),
           pl.BlockSpec(W1.shape, lambda i: (0, 0))]
        + ([pl.BlockSpec((bn, HID), lambda i: (i, 0))] if has_res else [])
    )
    args = list(xs) + list(Ws) + list(adds) + [vecs, W1] + ([res] if has_res else [])
    return pl.pallas_call(
        body,
        grid=(grid,),
        in_specs=in_specs,
        out_specs=pl.BlockSpec((bn, dout), lambda i: (i, 0)),
        out_shape=jax.ShapeDtypeStruct((n, dout), jnp.float32),
    )(*args)


def _vecs(p, dout=HID):
    v = jnp.zeros((8, HID), jnp.float32)
    v = v.at[0, :].set(p["b0"])
    v = v.at[1, :dout].set(p["b1"])
    v = v.at[2, :dout].set(p["gamma"])
    v = v.at[3, :dout].set(p["beta"])
    return v


def _mlp_small(p, x, bn):
    w0 = _pad_rows(p["W0"], 128)
    return _mlp_tc([x], [w0], [], _vecs(p), p["W1"], None,
                   ln=True, bn=bn, dout=HID)


def _proj_tc(x, W):
    n = x.shape[0]
    bn = next(b for b in (512, 384, 256, 128) if n % b == 0)

    def body(x_ref, w_ref, out_ref):
        out_ref[...] = jnp.dot(x_ref[...], w_ref[...],
                               preferred_element_type=jnp.float32)

    return pl.pallas_call(
        body,
        grid=(n // bn,),
        in_specs=[pl.BlockSpec((bn, HID), lambda i: (i, 0)),
                  pl.BlockSpec(W.shape, lambda i: (0, 0))],
        out_specs=pl.BlockSpec((bn, HID), lambda i: (i, 0)),
        out_shape=jax.ShapeDtypeStruct((n, HID), jnp.float32),
    )(x, W)


@functools.lru_cache(maxsize=None)
def _sc_gather_pair_fn(B, V1, V2):
    per_w = B // NW
    mesh = plsc.VectorSubcoreMesh(core_axis_name="c", subcore_axis_name="s")

    GC = 40
    k = per_w // GC

    @functools.partial(
        pl.kernel,
        out_type=[jax.ShapeDtypeStruct((B, HID), jnp.float32),
                  jax.ShapeDtypeStruct((B, HID), jnp.float32)],
        mesh=mesh,
        scratch_types=[
            pltpu.VMEM((per_w,), jnp.int32),
            pltpu.VMEM((per_w,), jnp.int32),
            pltpu.VMEM((2, GC, HID), jnp.float32),
            pltpu.VMEM((2, GC, HID), jnp.float32),
            pltpu.SemaphoreType.DMA,
            pltpu.SemaphoreType.DMA,
            pltpu.SemaphoreType.DMA,
            pltpu.SemaphoreType.DMA,
            pltpu.SemaphoreType.DMA,
            pltpu.SemaphoreType.DMA,
            pltpu.SemaphoreType.DMA,
            pltpu.SemaphoreType.DMA,
        ],
    )
    def kern(tab1, idx1, tab2, idx2, out1, out2, idx1_v, idx2_v,
             rows1_v, rows2_v, g1a, g1b, g2a, g2b, o1a, o1b, o2a, o2b):
        wid = lax.axis_index("s") * NC + lax.axis_index("c")
        base = wid * per_w
        pltpu.sync_copy(idx1.at[pl.ds(base, per_w)], idx1_v)
        pltpu.sync_copy(idx2.at[pl.ds(base, per_w)], idx2_v)
        gsems = ((g1a, g2a), (g1b, g2b))
        osems = ((o1a, o2a), (o1b, o2b))
        ocps = {}
        for i in range(k):
            b = i & 1
            if i >= 2:
                ocps[(i - 2, 0)].wait()
                ocps[(i - 2, 1)].wait()
            cp1 = pltpu.async_copy(
                tab1.at[idx1_v.at[pl.ds(i * GC, GC)]],
                rows1_v.at[b], gsems[b][0])
            cp2 = pltpu.async_copy(
                tab2.at[idx2_v.at[pl.ds(i * GC, GC)]],
                rows2_v.at[b], gsems[b][1])
            cp1.wait()
            cp2.wait()
            ocps[(i, 0)] = pltpu.async_copy(
                rows1_v.at[b], out1.at[pl.ds(base + i * GC, GC)],
                osems[b][0])
            ocps[(i, 1)] = pltpu.async_copy(
                rows2_v.at[b], out2.at[pl.ds(base + i * GC, GC)],
                osems[b][1])
        for i in range(max(k - 2, 0), k):
            ocps[(i, 0)].wait()
            ocps[(i, 1)].wait()

    return kern


def _sc_gather_pair(tab1, idx1, tab2, idx2):
    return _sc_gather_pair_fn(idx1.shape[0], tab1.shape[0], tab2.shape[0])(
        tab1, idx1, tab2, idx2)


LCAP = 4096
GB = 64


@functools.lru_cache(maxsize=None)
def _sc_segplan_fn(E, TR, nch):
    n_idxc = E // IDXC
    TRP = ((TR + 31) // 16) * 16
    CNP = nch * NW * 8 + 16
    mesh = plsc.VectorSubcoreMesh(core_axis_name="c", subcore_axis_name="s")

    @functools.partial(
        pl.kernel,
        out_type=[jax.ShapeDtypeStruct((nch * NW * LCAP,), jnp.int32),
                  jax.ShapeDtypeStruct((nch * NW * LCAP,), jnp.int32),
                  jax.ShapeDtypeStruct((CNP,), jnp.int32)],
        mesh=mesh,
        scratch_types=[
            pltpu.VMEM((IDXC,), jnp.int32),
            pltpu.VMEM((LCAP,), jnp.int32),
            pltpu.VMEM((LCAP,), jnp.int32),
            pltpu.VMEM((LCAP,), jnp.int32),
            pltpu.VMEM((LCAP,), jnp.int32),
            pltpu.VMEM((TRP,), jnp.int32),
            pltpu.VMEM((TRP,), jnp.int32),
        ],
        compiler_params=pltpu.CompilerParams(needs_layout_passes=False),
    )
    def kern(idx, elist, ctrl, cnts, idx_v, e_v, d_v, se_v, sc_v, cnt_v,
             offs_v):
        t = lax.axis_index("s") * NC + lax.axis_index("c")
        iota = lax.iota(jnp.int32, 16)
        lane0 = iota == 0
        zero16 = jnp.zeros((16,), jnp.int32)

        def put1(ref, pos, val):
            plsc.store_compressed(ref.at[pl.ds(pos, 16)],
                                  jnp.full((16,), val, jnp.int32), mask=lane0)

        for ch in range(nch):
            lo = (ch * NW + t) * TR
            for q in range(TRP // 16):
                cnt_v[pl.ds(q * 16, 16)] = zero16

            def idx_chunk(ci, cur):
                pltpu.sync_copy(idx.at[pl.ds(ci * IDXC, IDXC)], idx_v)

                def grp(gi, cur):
                    dst16 = idx_v[pl.ds(gi * 16, 16)]
                    m = (dst16 >= lo) & (dst16 < lo + TR)
                    c_ = jnp.sum(jnp.where(m, 1.0, 0.0)).astype(jnp.int32)
                    plsc.store_compressed(e_v.at[pl.ds(cur, 16)],
                                          ci * IDXC + gi * 16 + iota, mask=m)
                    plsc.store_compressed(d_v.at[pl.ds(cur, 16)],
                                          dst16 - lo, mask=m)
                    return cur + c_

                return lax.fori_loop(0, IDXC // 16, grp, cur)

            cur = lax.fori_loop(0, n_idxc, idx_chunk, jnp.int32(0))

            def count(i, _):
                d = d_v[pl.ds(i, 16)][0]
                put1(cnt_v, d, cnt_v[pl.ds(d, 16)][0] + 1)
                return 0

            lax.fori_loop(0, cur, count, 0)

            put1(offs_v, 0, jnp.int32(0))

            def pfx(d, acc):
                acc = acc + cnt_v[pl.ds(d, 16)][0]
                put1(offs_v, d + 1, acc)
                return acc

            lax.fori_loop(0, TR, pfx, jnp.int32(0))

            for q in range(TRP // 16):
                cnt_v[pl.ds(q * 16, 16)] = offs_v[pl.ds(q * 16, 16)]

            def place(i, _):
                d = d_v[pl.ds(i, 16)][0]
                p = cnt_v[pl.ds(d, 16)][0]
                put1(se_v, p, e_v[pl.ds(i, 16)][0])
                put1(sc_v, p, d * 2)
                put1(cnt_v, d, p + 1)
                return 0

            lax.fori_loop(0, cur, place, 0)

            def flag(d, _):
                o = offs_v[pl.ds(d, 16)]

                @pl.when(o[1] > o[0])
                def _():
                    put1(sc_v, o[1] - 1, sc_v[pl.ds(o[1] - 1, 16)][0] + 1)

                return 0

            lax.fori_loop(0, TR, flag, 0)

            for q in range(GB // 16 + 1):
                plsc.store_compressed(se_v.at[pl.ds(cur + q * 16, 16)],
                                      zero16, mask=iota >= 0)

            base = (ch * NW + t) * LCAP
            pltpu.sync_copy(se_v, elist.at[pl.ds(base, LCAP)])
            pltpu.sync_copy(sc_v, ctrl.at[pl.ds(base, LCAP)])
            pltpu.sync_copy(offs_v.at[pl.ds(TR, 8)],
                            cnts.at[pl.ds((ch * NW + t) * 8, 8)])

    return kern


def _sc_segplan(idx, TR, nch):
    return _sc_segplan_fn(idx.shape[0], TR, nch)(idx)


@functools.lru_cache(maxsize=None)
def _sc_segexec_fn(E, TR, nch):
    CNP = nch * NW * 8 + 16
    mesh = plsc.VectorSubcoreMesh(core_axis_name="c", subcore_axis_name="s")

    @functools.partial(
        pl.kernel,
        out_type=jax.ShapeDtypeStruct((nch * NW * TR, HID), jnp.float32),
        mesh=mesh,
        scratch_types=[
            pltpu.VMEM((LCAP,), jnp.int32),
            pltpu.VMEM((LCAP,), jnp.int32),
            pltpu.VMEM((CNP,), jnp.int32),
            pltpu.VMEM((GB, HID), jnp.float32),
            pltpu.VMEM((TR, HID), jnp.float32),
            pltpu.SemaphoreType.DMA,
        ],
        compiler_params=pltpu.CompilerParams(needs_layout_passes=False),
    )
    def kern(vals, elist, ctrl, cnts, out, se_v, sc_v, cn_v, rows_v, ob_v,
             sem):
        t = lax.axis_index("s") * NC + lax.axis_index("c")
        zf = jnp.zeros((16,), jnp.float32)
        pltpu.sync_copy(cnts, cn_v)

        for ch in range(nch):
            base = ch * NW + t
            pltpu.sync_copy(elist.at[pl.ds(base * LCAP, LCAP)], se_v)
            pltpu.sync_copy(ctrl.at[pl.ds(base * LCAP, LCAP)], sc_v)
            cnt = cn_v[pl.ds((ch * NW + t) * 8, 16)][0]

            def zrow(d, _):
                for j in range(HID // 16):
                    ob_v[d, pl.ds(j * 16, 16)] = zf
                return 0

            lax.fori_loop(0, TR, zrow, 0)

            def batch(b, accs):
                pltpu.async_copy(vals.at[se_v.at[pl.ds(b * GB, GB)]],
                                 rows_v, sem).wait()
                rem = jnp.minimum(cnt - b * GB, GB)

                def row(i, accs):
                    cw = sc_v[pl.ds(b * GB + i, 16)][0]
                    accs = tuple(a + rows_v[i, pl.ds(j * 16, 16)]
                                 for j, a in enumerate(accs))
                    d = lax.shift_right_logical(cw, 1)

                    def dump(ac):
                        for j in range(HID // 16):
                            ob_v[d, pl.ds(j * 16, 16)] = ac[j]
                        return tuple(zf for _ in range(HID // 16))

                    return lax.cond(jnp.bitwise_and(cw, 1) == 1,
                                    dump, lambda ac: ac, accs)

                return lax.fori_loop(0, rem, row, accs)

            accs0 = tuple(zf for _ in range(HID // 16))
            nb = lax.shift_right_logical(cnt + GB - 1, 6)
            lax.fori_loop(0, nb, batch, accs0)
            pltpu.sync_copy(ob_v, out.at[pl.ds(base * TR, TR)])

    return kern


def _sc_segexec(vals, plan, TR, nch):
    elist, ctrl, cnts = plan
    return _sc_segexec_fn(vals.shape[0], TR, nch)(vals, elist, ctrl, cnts)


def _edge_stage(p, e, src_tab, src_idx, dst_tab, dst_idx, bn):
    W0 = p["W0"]
    W0e, W0s, W0d = W0[:HID], W0[HID:2 * HID], W0[2 * HID:]
    ps = _proj_tc(src_tab, W0s)
    pd = _proj_tc(dst_tab, W0d)
    gs, gd = _sc_gather_pair(ps, src_idx, pd, dst_idx)
    return _mlp_tc([e], [W0e], [gs, gd], _vecs(p), p["W1"], e,
                   ln=True, bn=bn, dout=HID)


def _node_stage(p, agg, x, bn):
    W0 = p["W0"]
    return _mlp_tc([agg, x], [W0[:HID], W0[HID:]], [], _vecs(p), p["W1"], x,
                   ln=True, bn=bn, dout=HID)


def kernel(grid_nfeat, mesh_ndata, g2m_edata, mesh_edata, m2g_edata, params,
           g2m_src, g2m_dst, mesh_src, mesh_dst, m2g_src, m2g_dst):
    n_grid, n_mesh = grid_nfeat.shape[0], mesh_ndata.shape[0]
    NGP = 10240
    NMP = 2688
    MTR = 88
    GTR = 160
    EG = 40960
    EM = 15360
    ED = 30720

    gridf = _pad2(grid_nfeat, NGP, 128)
    meshf = _pad2(mesh_ndata, NMP, 128)
    g2mf = _pad2(g2m_edata, EG, 128)
    meshef = _pad2(mesh_edata, EM, 128)
    m2gf = _pad2(m2g_edata, ED, 128)

    g2m_src_g = _pad_idx(g2m_src, EG, 0)
    g2m_dst_g = _pad_idx(g2m_dst, EG, 0)
    g2m_dst_s = _pad_idx(g2m_dst, EG, _BIG)

    mesh_src_g = _pad_idx(mesh_src, EM, 0)
    mesh_dst_g = _pad_idx(mesh_dst, EM, 0)
    mesh_dst_s = _pad_idx(mesh_dst, EM, _BIG)
    m2g_src_g = _pad_idx(m2g_src, ED, 0)
    m2g_dst_g = _pad_idx(m2g_dst, ED, 0)
    m2g_dst_s = _pad_idx(m2g_dst, ED, _BIG)

    plan_g2m = _sc_segplan(g2m_dst_s, MTR, 1)
    plan_mesh = _sc_segplan(mesh_dst_s, MTR, 1)
    plan_m2g = _sc_segplan(m2g_dst_s, GTR, 2)

    grid = _mlp_small(params["emb_grid"], gridf, 512)
    mesh = _mlp_small(params["emb_mesh"], meshf, 384)
    e_g2m = _mlp_small(params["emb_g2m"], g2mf, 512)
    e_mesh = _mlp_small(params["emb_meshe"], meshef, 512)

    e_g2m = _edge_stage(params["enc_edge"], e_g2m, grid, g2m_src_g,
                        mesh, g2m_dst_g, 512)
    agg = _sc_segexec(e_g2m, plan_g2m, MTR, 1)[:NMP]
    mesh = _node_stage(params["enc_mesh"], agg, mesh, 384)
    grid_enc = _mlp_tc([grid], [params["enc_grid"]["W0"]], [],
                       _vecs(params["enc_grid"]), params["enc_grid"]["W1"],
                       grid, ln=True, bn=512, dout=HID)

    for blk in params["proc"]:
        e_mesh = _edge_stage(blk["edge"], e_mesh, mesh, mesh_src_g,
                             mesh, mesh_dst_g, 512)
        agg = _sc_segexec(e_mesh, plan_mesh, MTR, 1)[:NMP]
        mesh = _node_stage(blk["node"], agg, mesh, 384)

    e_m2g = _mlp_small(params["dec_emb"], m2gf, 512)
    e_m2g = _edge_stage(params["dec_edge"], e_m2g, mesh, m2g_src_g,
                        grid_enc, m2g_dst_g, 512)
    agg = _sc_segexec(e_m2g, plan_m2g, GTR, 2)
    grid_dec = _node_stage(params["dec_node"], agg, grid_enc, 512)

    pf = params["finale"]
    w1 = jnp.pad(pf["W1"], ((0, 0), (0, 127)))
    vec = jnp.zeros((8, HID), jnp.float32).at[0, :].set(pf["b0"])
    vec = vec.at[1, 0].set(pf["b1"][0])
    out = _mlp_tc([grid_dec], [pf["W0"]], [], vec, w1, None,
                  ln=False, bn=512, dout=128)
    return out[:n_grid, :1]
